# baseline (speedup 1.0000x reference)
import functools

import jax
import jax.numpy as jnp
from jax import lax
from jax.experimental import pallas as pl
from jax.experimental.pallas import tpu as pltpu
from jax.experimental.pallas import tpu_sc as plsc

VOCAB = 1000000
N_EMBD = 64
BLOCK = 256
B = 1024
T = 256

NC = 2
NS = 16
NW = NC * NS

ROWS = B * T
R_PER_W = ROWS // NW
CHUNK = 128
NCHUNK = R_PER_W // CHUNK


def _emb_body(idx_hbm, tok_hbm, pos_hbm, out_hbm, idx_v, pos_v, rows_v, sem):
    wid = lax.axis_index("s") * NC + lax.axis_index("c")
    base = wid * R_PER_W

    pltpu.sync_copy(idx_hbm.at[wid], idx_v)
    pltpu.sync_copy(pos_hbm, pos_v)

    def chunk_body(j, _):
        t0 = (j % (BLOCK // CHUNK)) * CHUNK
        pltpu.async_copy(tok_hbm.at[idx_v.at[j]], rows_v, sem).wait()

        def add_row(r, _):
            for q in range(N_EMBD // 16):
                sl = pl.ds(q * 16, 16)
                rows_v[r, sl] = rows_v[r, sl] + pos_v[t0 + r, sl]
            return 0

        lax.fori_loop(0, CHUNK, add_row, 0, unroll=2)
        pltpu.sync_copy(rows_v, out_hbm.at[pl.ds(base + j * CHUNK, CHUNK)])
        return 0

    lax.fori_loop(0, NCHUNK, chunk_body, 0)


@jax.jit
def _emb_call(idx, tok_table, pos_table):
    mesh = plsc.VectorSubcoreMesh(
        core_axis_name="c", subcore_axis_name="s", num_cores=NC, num_subcores=NS
    )
    return pl.kernel(
        _emb_body,
        out_type=jax.ShapeDtypeStruct((ROWS, N_EMBD), jnp.float32),
        mesh=mesh,
        scratch_types=[
            pltpu.VMEM((NCHUNK, CHUNK), jnp.int32),
            pltpu.VMEM((BLOCK, N_EMBD), jnp.float32),
            pltpu.VMEM((CHUNK, N_EMBD), jnp.float32),
            pltpu.SemaphoreType.DMA,
        ],
        compiler_params=pltpu.CompilerParams(use_tc_tiling_on_sc=False),
    )(idx, tok_table, pos_table)


def kernel(x, tok_table, pos_table):
    idx = x.astype(jnp.int32).reshape(NW, NCHUNK, CHUNK)
    out = _emb_call(idx, tok_table, pos_table)
    return out.reshape(B, T, N_EMBD)


# speedup vs baseline: 1.2054x; 1.2054x over previous
import jax
import jax.numpy as jnp
from jax import lax
from jax.experimental import pallas as pl
from jax.experimental.pallas import tpu as pltpu
from jax.experimental.pallas import tpu_sc as plsc

VOCAB = 1000000
N_EMBD = 64
BLOCK = 256
B = 1024
T = 256

NC = 2
NS = 16
NW = NC * NS

ROWS = B * T
R_PER_W = ROWS // NW
CHUNK = 128
NCHUNK = R_PER_W // CHUNK
NSLOT = 8
NGROUP = NCHUNK // NSLOT
LEAD = NSLOT // 2


def _emb_body(idx_hbm, tok_hbm, pos_hbm, out_hbm, idx_v, pos_v, rows_v, *sems):
    g_sem = sems[:NSLOT]
    o_sem = sems[NSLOT:]
    wid = lax.axis_index("s") * NC + lax.axis_index("c")
    base = wid * R_PER_W

    pltpu.sync_copy(idx_hbm.at[wid], idx_v)
    pltpu.sync_copy(pos_hbm, pos_v)

    def start_gather(c, s):
        pltpu.async_copy(tok_hbm.at[idx_v.at[c]], rows_v.at[s], g_sem[s])

    def wait_gather(s):
        pltpu.make_async_copy(tok_hbm.at[idx_v.at[0]], rows_v.at[s], g_sem[s]).wait()

    def start_out(c, s):
        pltpu.async_copy(
            rows_v.at[s], out_hbm.at[pl.ds(base + c * CHUNK, CHUNK)], o_sem[s]
        )

    def wait_out(s):
        pltpu.make_async_copy(
            rows_v.at[s], out_hbm.at[pl.ds(base, CHUNK)], o_sem[s]
        ).wait()

    def add_pos(s, t0):
        def add_row(r, _):
            for q in range(N_EMBD // 16):
                sl = pl.ds(q * 16, 16)
                rows_v[s, r, sl] = rows_v[s, r, sl] + pos_v[t0 + r, sl]
            return 0

        lax.fori_loop(0, CHUNK, add_row, 0, unroll=2)

    for b in range(LEAD):
        start_gather(b, b)

    for b in range(NSLOT):
        wait_gather(b)
        add_pos(b, (b % 2) * CHUNK)
        start_out(b, b)
        s = (b + LEAD) % NSLOT
        if b >= LEAD:
            wait_out(s)
        start_gather(b + LEAD, s)

    def group(jo, _):
        for b in range(NSLOT):
            c = jo * NSLOT + b
            wait_gather(b)
            add_pos(b, (b % 2) * CHUNK)
            start_out(c, b)
            s = (b + LEAD) % NSLOT
            wait_out(s)
            start_gather(c + LEAD, s)
        return 0

    lax.fori_loop(1, NGROUP - 1, group, 0)

    for b in range(NSLOT):
        c = (NGROUP - 1) * NSLOT + b
        wait_gather(b)
        add_pos(b, (b % 2) * CHUNK)
        start_out(c, b)
        if b < LEAD:
            s = b + LEAD
            wait_out(s)
            start_gather(c + LEAD, s)

    for b in range(NSLOT):
        wait_out(b)


@jax.jit
def _emb_call(idx, tok_table, pos_table):
    mesh = plsc.VectorSubcoreMesh(
        core_axis_name="c", subcore_axis_name="s", num_cores=NC, num_subcores=NS
    )
    return pl.kernel(
        _emb_body,
        out_type=jax.ShapeDtypeStruct((ROWS, N_EMBD), jnp.float32),
        mesh=mesh,
        scratch_types=[
            pltpu.VMEM((NCHUNK, CHUNK), jnp.int32),
            pltpu.VMEM((BLOCK, N_EMBD), jnp.float32),
            pltpu.VMEM((NSLOT, CHUNK, N_EMBD), jnp.float32),
        ]
        + [pltpu.SemaphoreType.DMA] * (2 * NSLOT),
        compiler_params=pltpu.CompilerParams(use_tc_tiling_on_sc=False),
    )(idx, tok_table, pos_table)


def kernel(x, tok_table, pos_table):
    idx = x.astype(jnp.int32).reshape(NW, NCHUNK, CHUNK)
    out = _emb_call(idx, tok_table, pos_table)
    return out.reshape(B, T, N_EMBD)
